# baseline (speedup 1.0000x reference)
import functools

import jax
import jax.numpy as jnp
from jax import lax
from jax.experimental import pallas as pl
from jax.experimental.pallas import tpu as pltpu
from jax.experimental.pallas import tpu_sc as plsc

_CH = 128


def _sc_info():
    info = plsc.get_sparse_core_info()
    return info.num_cores, info.num_subcores


def _worker_rows(w, nch, nw):
    extra = nch % nw
    base_rows = nch // nw
    start = w * base_rows + jnp.minimum(w, extra)
    return start, base_rows, w < extra


def _sc_gather(x, src2d):
    n, d = x.shape
    nch, ch = src2d.shape
    e = nch * ch
    nc, ns = _sc_info()
    nw = nc * ns
    base_rows = nch // nw
    nrounds = (base_rows + 10) // 10
    mesh = plsc.VectorSubcoreMesh(core_axis_name="c", subcore_axis_name="s")

    @functools.partial(
        pl.kernel,
        out_type=jax.ShapeDtypeStruct((e, d), jnp.float32),
        mesh=mesh,
        compiler_params=pltpu.CompilerParams(use_tc_tiling_on_sc=False),
        scratch_types=[
            pltpu.VMEM((base_rows + 1, ch), jnp.int32),
            pltpu.VMEM((10 * ch, d), jnp.float32),
            pltpu.VMEM((10 * ch, d), jnp.float32),
            pltpu.SemaphoreType.DMA,
            pltpu.SemaphoreType.DMA,
        ],
    )
    def gather_k(x_hbm, src_hbm, out_hbm, idx_v, rows0, rows1, gsem, wsem):
        c = lax.axis_index("c")
        s = lax.axis_index("s")
        w = s * nc + c
        start, _, has_extra = _worker_rows(w, nch, nw)
        pltpu.sync_copy(src_hbm.at[pl.ds(start, base_rows)],
                        idx_v.at[pl.ds(0, base_rows)])

        @pl.when(has_extra)
        def _():
            pltpu.sync_copy(src_hbm.at[pl.ds(start + base_rows, 1)],
                            idx_v.at[pl.ds(base_rows, 1)])

        rows = (rows0, rows1)
        wb = {}
        for r in range(nrounds):
            buf = rows[r % 2]
            if r >= 2:
                for dsc in wb[r - 2]:
                    dsc.wait()
            jlo = r * 10
            jhi = min(jlo + 10, base_rows)
            descs = []
            for j in range(jlo, jhi):
                descs.append(pltpu.async_copy(
                    x_hbm.at[idx_v.at[j]],
                    buf.at[pl.ds((j - jlo) * ch, ch)], gsem))
            if jhi == base_rows:
                @pl.when(has_extra)
                def _():
                    pltpu.async_copy(
                        x_hbm.at[idx_v.at[base_rows]],
                        buf.at[pl.ds((base_rows - jlo) * ch, ch)],
                        gsem).wait()
            for dsc in descs:
                dsc.wait()
            nfull = jhi - jlo
            wbs = [pltpu.async_copy(
                buf.at[pl.ds(0, nfull * ch)],
                out_hbm.at[pl.ds((start + jlo) * ch, nfull * ch)], wsem)]
            if jhi == base_rows:
                @pl.when(has_extra)
                def _():
                    pltpu.async_copy(
                        buf.at[pl.ds(nfull * ch, ch)],
                        out_hbm.at[pl.ds((start + base_rows) * ch, ch)],
                        wsem).wait()
            wb[r] = wbs
        for r in (nrounds - 2, nrounds - 1):
            if r >= 0:
                for dsc in wb[r]:
                    dsc.wait()

    return gather_k(x, src2d)


def _sc_scatter(msg, dst2d, n):
    e, d = msg.shape
    nch, ch = dst2d.shape
    assert nch * ch == e
    nc, ns = _sc_info()
    nw = nc * ns
    base_rows = nch // nw
    nrounds = (base_rows + 10) // 10
    mesh = plsc.VectorSubcoreMesh(core_axis_name="c", subcore_axis_name="s")

    z32 = jnp.zeros((n, d), jnp.float32)
    z16 = jnp.zeros((n, 16), jnp.float32)
    ones = jnp.ones((ch, 16), jnp.float32)

    @functools.partial(
        pl.kernel,
        out_type=(
            jax.ShapeDtypeStruct((nc, n, d), jnp.float32),
            jax.ShapeDtypeStruct((nc, n, 16), jnp.float32),
        ),
        mesh=mesh,
        compiler_params=pltpu.CompilerParams(use_tc_tiling_on_sc=False),
        scratch_types=[
            pltpu.VMEM((base_rows + 1, ch), jnp.int32),
            pltpu.VMEM((10 * ch, d), jnp.float32),
            pltpu.VMEM((10 * ch, d), jnp.float32),
            pltpu.VMEM((ch, 16), jnp.float32),
            pltpu.VMEM_SHARED((n, d), jnp.float32),
            pltpu.VMEM_SHARED((n, 16), jnp.float32),
            pltpu.SemaphoreType.DMA,
        ],
    )
    def scatter_k(msg_hbm, dst_hbm, z32_hbm, z16_hbm, ones_hbm,
                  agg_hbm, deg_hbm, idx_v, rows0, rows1, ones_v,
                  agg_sh, deg_sh, asem):
        c = lax.axis_index("c")
        s = lax.axis_index("s")
        w = s * nc + c
        start, _, has_extra = _worker_rows(w, nch, nw)

        @pl.when(s == 0)
        def _init():
            pltpu.sync_copy(z32_hbm, agg_sh)
            pltpu.sync_copy(z16_hbm, deg_sh)

        pltpu.sync_copy(ones_hbm, ones_v)
        pltpu.sync_copy(dst_hbm.at[pl.ds(start, base_rows)],
                        idx_v.at[pl.ds(0, base_rows)])

        @pl.when(has_extra)
        def _():
            pltpu.sync_copy(dst_hbm.at[pl.ds(start + base_rows, 1)],
                            idx_v.at[pl.ds(base_rows, 1)])

        plsc.subcore_barrier()

        rows = (rows0, rows1)
        adds = {}
        for r in range(nrounds):
            buf = rows[r % 2]
            if r >= 2:
                for dsc in adds[r - 2]:
                    dsc.wait()
            jlo = r * 10
            jhi = min(jlo + 10, base_rows)
            nfull = jhi - jlo
            pltpu.sync_copy(
                msg_hbm.at[pl.ds((start + jlo) * ch, nfull * ch)],
                buf.at[pl.ds(0, nfull * ch)])
            descs = []
            for j in range(jlo, jhi):
                descs.append(pltpu.async_copy(
                    buf.at[pl.ds((j - jlo) * ch, ch)],
                    agg_sh.at[idx_v.at[j]], asem, add=True))
                descs.append(pltpu.async_copy(
                    ones_v, deg_sh.at[idx_v.at[j]], asem, add=True))
            if jhi == base_rows:
                @pl.when(has_extra)
                def _():
                    pltpu.sync_copy(
                        msg_hbm.at[pl.ds((start + base_rows) * ch, ch)],
                        buf.at[pl.ds(nfull * ch, ch)])
                    pltpu.sync_copy(buf.at[pl.ds(nfull * ch, ch)],
                                    agg_sh.at[idx_v.at[base_rows]], add=True)
                    pltpu.sync_copy(ones_v,
                                    deg_sh.at[idx_v.at[base_rows]], add=True)
            adds[r] = descs
        for r in (nrounds - 2, nrounds - 1):
            if r >= 0:
                for dsc in adds[r]:
                    dsc.wait()

        plsc.subcore_barrier()

        @pl.when(s == 0)
        def _emit():
            pltpu.sync_copy(agg_sh, agg_hbm.at[c])
            pltpu.sync_copy(deg_sh, deg_hbm.at[c])

    return scatter_k(msg, dst2d, z32, z16, ones)


def _tc_messages(ea, xj, w1t, b1, w2t, b2):
    e, k_in = ea.shape
    d = xj.shape[1]
    hidden = w1t.shape[1]
    be = 1600
    assert e % be == 0
    grid = (e // be,)
    rep = (jnp.arange(d)[:, None] == (jnp.arange(d * d)[None, :] // d))
    rep = rep.astype(jnp.bfloat16)
    b2mat = b2.reshape(d, d).astype(jnp.float32)

    def body(ea_ref, xj_ref, w1_ref, b1_ref, w2_ref, b2_ref, rep_ref,
             out_ref):
        h = jnp.maximum(
            jnp.dot(ea_ref[...].astype(jnp.bfloat16), w1_ref[...],
                    preferred_element_type=jnp.float32) + b1_ref[...], 0.0)
        we = jnp.dot(h.astype(jnp.bfloat16), w2_ref[...],
                     preferred_element_type=jnp.float32)
        xjr = jnp.dot(xj_ref[...].astype(jnp.bfloat16), rep_ref[...],
                      preferred_element_type=jnp.float32)
        acc = jnp.zeros((be, 128), jnp.float32)
        for j in range(d * d // 128):
            acc = acc + (xjr[:, j * 128:(j + 1) * 128]
                         * we[:, j * 128:(j + 1) * 128])
        msg = jnp.dot(xj_ref[...], b2_ref[...],
                      preferred_element_type=jnp.float32)
        msg = msg + ((acc[:, 0:d] + acc[:, d:2 * d])
                     + (acc[:, 2 * d:3 * d] + acc[:, 3 * d:4 * d]))
        out_ref[...] = msg

    return pl.pallas_call(
        body,
        grid=grid,
        in_specs=[
            pl.BlockSpec((be, k_in), lambda i: (i, 0)),
            pl.BlockSpec((be, d), lambda i: (i, 0)),
            pl.BlockSpec((k_in, hidden), lambda i: (0, 0)),
            pl.BlockSpec((1, hidden), lambda i: (0, 0)),
            pl.BlockSpec((hidden, d * d), lambda i: (0, 0)),
            pl.BlockSpec((d, d), lambda i: (0, 0)),
            pl.BlockSpec((d, d * d), lambda i: (0, 0)),
        ],
        out_specs=pl.BlockSpec((be, d), lambda i: (i, 0)),
        out_shape=jax.ShapeDtypeStruct((e, d), jnp.float32),
    )(ea.astype(jnp.float32), xj, w1t.astype(jnp.bfloat16),
      b1.astype(jnp.float32), w2t.astype(jnp.bfloat16), b2mat,
      rep)


def _tc_finalize(x, agg_a, agg_b, deg_a, deg_b, batch2, root, conv_b,
                 gn_w, gn_b, gn_ms, num_graphs, eps):
    n, d = x.shape
    bn = 1000
    assert n % bn == 0
    grid = (n // bn,)

    def stats_body(x_ref, agg_a_ref, agg_b_ref, deg_a_ref, deg_b_ref,
                   bat_ref, root_ref, cb_ref, h2_ref, cnt_ref, s_ref, ss_ref):
        @pl.when(pl.program_id(0) == 0)
        def _init():
            cnt_ref[...] = jnp.zeros_like(cnt_ref)
            s_ref[...] = jnp.zeros_like(s_ref)
            ss_ref[...] = jnp.zeros_like(ss_ref)

        x = x_ref[...]
        agg_sum = ((agg_a_ref[0] + agg_a_ref[1])
                   + (agg_b_ref[0] + agg_b_ref[1]))
        deg = ((deg_a_ref[0][:, 0:1] + deg_a_ref[1][:, 0:1])
               + (deg_b_ref[0][:, 0:1] + deg_b_ref[1][:, 0:1]))
        agg = agg_sum / jnp.maximum(deg, 1.0)
        conv = agg + jnp.dot(x, root_ref[...],
                             preferred_element_type=jnp.float32) + cb_ref[...]
        h2 = jnp.maximum(conv, 0.0)
        h2_ref[...] = h2
        bat = bat_ref[...]
        oh = (bat == lax.broadcasted_iota(jnp.int32, (1, num_graphs), 1))
        oh = oh.astype(jnp.float32)
        ones_col = jnp.ones((bn, 1), jnp.float32)
        hcat = jnp.concatenate([h2, h2 * h2, jnp.broadcast_to(ones_col, (bn, d))],
                               axis=1)
        acc = lax.dot_general(oh, hcat, (((0,), (0,)), ((), ())),
                              preferred_element_type=jnp.float32)
        s_ref[...] += acc[:, 0:d]
        ss_ref[...] += acc[:, d:2 * d]
        cnt_ref[...] += acc[:, 2 * d:3 * d]

    h2, cnts, sums, sqs = pl.pallas_call(
        stats_body,
        grid=grid,
        in_specs=[
            pl.BlockSpec((bn, d), lambda i: (i, 0)),
            pl.BlockSpec((2, bn, d), lambda i: (0, i, 0)),
            pl.BlockSpec((2, bn, d), lambda i: (0, i, 0)),
            pl.BlockSpec((2, bn, 16), lambda i: (0, i, 0)),
            pl.BlockSpec((2, bn, 16), lambda i: (0, i, 0)),
            pl.BlockSpec((bn, 1), lambda i: (i, 0)),
            pl.BlockSpec((d, d), lambda i: (0, 0)),
            pl.BlockSpec((1, d), lambda i: (0, 0)),
        ],
        out_specs=[
            pl.BlockSpec((bn, d), lambda i: (i, 0)),
            pl.BlockSpec((num_graphs, d), lambda i: (0, 0)),
            pl.BlockSpec((num_graphs, d), lambda i: (0, 0)),
            pl.BlockSpec((num_graphs, d), lambda i: (0, 0)),
        ],
        out_shape=[
            jax.ShapeDtypeStruct((n, d), jnp.float32),
            jax.ShapeDtypeStruct((num_graphs, d), jnp.float32),
            jax.ShapeDtypeStruct((num_graphs, d), jnp.float32),
            jax.ShapeDtypeStruct((num_graphs, d), jnp.float32),
        ],
    )(x, agg_a, agg_b, deg_a, deg_b, batch2, root, conv_b)

    def norm_body(h2_ref, x_ref, bat_ref, cnt_ref, s_ref, ss_ref,
                  gw_ref, gb_ref, gms_ref, out_ref):
        cnt = jnp.maximum(cnt_ref[...], 1.0)
        gms = gms_ref[...]
        mean = s_ref[...] / cnt
        var = ss_ref[...] / cnt - mean * mean * gms * (2.0 - gms)
        bat = bat_ref[...]
        h2 = h2_ref[...]
        oh = (bat == lax.broadcasted_iota(jnp.int32, (1, num_graphs), 1))
        oh = oh.astype(jnp.float32)
        mean_b = jnp.dot(oh, mean, preferred_element_type=jnp.float32)
        var_b = jnp.dot(oh, var, preferred_element_type=jnp.float32)
        cen = h2 - mean_b * gms
        normed = gw_ref[...] * cen * lax.rsqrt(var_b + eps) + gb_ref[...]
        out_ref[...] = normed + x_ref[...]

    return pl.pallas_call(
        norm_body,
        grid=grid,
        in_specs=[
            pl.BlockSpec((bn, d), lambda i: (i, 0)),
            pl.BlockSpec((bn, d), lambda i: (i, 0)),
            pl.BlockSpec((bn, 1), lambda i: (i, 0)),
            pl.BlockSpec((num_graphs, d), lambda i: (0, 0)),
            pl.BlockSpec((num_graphs, d), lambda i: (0, 0)),
            pl.BlockSpec((num_graphs, d), lambda i: (0, 0)),
            pl.BlockSpec((1, d), lambda i: (0, 0)),
            pl.BlockSpec((1, d), lambda i: (0, 0)),
            pl.BlockSpec((1, d), lambda i: (0, 0)),
        ],
        out_specs=pl.BlockSpec((bn, d), lambda i: (i, 0)),
        out_shape=jax.ShapeDtypeStruct((n, d), jnp.float32),
    )(h2, x, batch2, cnts, sums, sqs, gn_w, gn_b, gn_ms)


def kernel(x, edge_index, edge_attr, batch, W1, b1, W2, b2, root, conv_bias,
           gn_weight, gn_bias, gn_mean_scale):
    n, d = x.shape
    num_graphs = 8
    eps = 1e-5
    e = edge_attr.shape[0]
    src2d = edge_index[0].reshape(e // _CH, _CH)
    dst2d = edge_index[1].reshape(e // _CH, _CH)
    hr = e // _CH // 2
    w1t = W1.T
    b1r = b1.reshape(1, -1)
    w2t = W2.T
    b2r = b2.reshape(1, -1)

    xj_a = _sc_gather(x, src2d[:hr])
    xj_b = _sc_gather(x, src2d[hr:])
    msg_a = _tc_messages(edge_attr[:hr * _CH], xj_a, w1t, b1r, w2t, b2r)
    agg_a, deg_a = _sc_scatter(msg_a, dst2d[:hr], n)
    msg_b = _tc_messages(edge_attr[hr * _CH:], xj_b, w1t, b1r, w2t, b2r)
    agg_b, deg_b = _sc_scatter(msg_b, dst2d[hr:], n)
    return _tc_finalize(x, agg_a, agg_b, deg_a, deg_b, batch.reshape(n, 1),
                        root, conv_bias.reshape(1, -1),
                        gn_weight.reshape(1, -1), gn_bias.reshape(1, -1),
                        gn_mean_scale.reshape(1, -1), num_graphs, eps)


# speedup vs baseline: 1.0858x; 1.0858x over previous
import functools

import jax
import jax.numpy as jnp
from jax import lax
from jax.experimental import pallas as pl
from jax.experimental.pallas import tpu as pltpu
from jax.experimental.pallas import tpu_sc as plsc

_CH = 128


def _sc_info():
    info = plsc.get_sparse_core_info()
    return info.num_cores, info.num_subcores


def _worker_rows(w, nch, nw):
    extra = nch % nw
    base_rows = nch // nw
    start = w * base_rows + jnp.minimum(w, extra)
    return start, base_rows, w < extra


def _sc_gather(x, src2d):
    n, d = x.shape
    nch, ch = src2d.shape
    e = nch * ch
    nc, ns = _sc_info()
    nw = nc * ns
    base_rows = nch // nw
    nrounds = (base_rows + 10) // 10
    mesh = plsc.VectorSubcoreMesh(core_axis_name="c", subcore_axis_name="s")

    @functools.partial(
        pl.kernel,
        out_type=jax.ShapeDtypeStruct((e, d), jnp.float32),
        mesh=mesh,
        compiler_params=pltpu.CompilerParams(use_tc_tiling_on_sc=False),
        scratch_types=[
            pltpu.VMEM((base_rows + 1, ch), jnp.int32),
            pltpu.VMEM((10 * ch, d), jnp.float32),
            pltpu.VMEM((10 * ch, d), jnp.float32),
            pltpu.SemaphoreType.DMA,
            pltpu.SemaphoreType.DMA,
        ],
    )
    def gather_k(x_hbm, src_hbm, out_hbm, idx_v, rows0, rows1, gsem, wsem):
        c = lax.axis_index("c")
        s = lax.axis_index("s")
        w = s * nc + c
        start, _, has_extra = _worker_rows(w, nch, nw)
        pltpu.sync_copy(src_hbm.at[pl.ds(start, base_rows)],
                        idx_v.at[pl.ds(0, base_rows)])

        @pl.when(has_extra)
        def _():
            pltpu.sync_copy(src_hbm.at[pl.ds(start + base_rows, 1)],
                            idx_v.at[pl.ds(base_rows, 1)])

        rows = (rows0, rows1)
        wb = {}
        for r in range(nrounds):
            buf = rows[r % 2]
            if r >= 2:
                for dsc in wb[r - 2]:
                    dsc.wait()
            jlo = r * 10
            jhi = min(jlo + 10, base_rows)
            descs = []
            for j in range(jlo, jhi):
                descs.append(pltpu.async_copy(
                    x_hbm.at[idx_v.at[j]],
                    buf.at[pl.ds((j - jlo) * ch, ch)], gsem))
            if jhi == base_rows:
                @pl.when(has_extra)
                def _():
                    pltpu.async_copy(
                        x_hbm.at[idx_v.at[base_rows]],
                        buf.at[pl.ds((base_rows - jlo) * ch, ch)],
                        gsem).wait()
            for dsc in descs:
                dsc.wait()
            nfull = jhi - jlo
            wbs = [pltpu.async_copy(
                buf.at[pl.ds(0, nfull * ch)],
                out_hbm.at[pl.ds((start + jlo) * ch, nfull * ch)], wsem)]
            if jhi == base_rows:
                @pl.when(has_extra)
                def _():
                    pltpu.async_copy(
                        buf.at[pl.ds(nfull * ch, ch)],
                        out_hbm.at[pl.ds((start + base_rows) * ch, ch)],
                        wsem).wait()
            wb[r] = wbs
        for r in (nrounds - 2, nrounds - 1):
            if r >= 0:
                for dsc in wb[r]:
                    dsc.wait()

    return gather_k(x, src2d)


def _sc_scatter(msg, dst2d, n):
    e, d = msg.shape
    nch, ch = dst2d.shape
    assert nch * ch == e
    nc, ns = _sc_info()
    nw = nc * ns
    base_rows = nch // nw
    nrounds = (base_rows + 10) // 10
    mesh = plsc.VectorSubcoreMesh(core_axis_name="c", subcore_axis_name="s")

    z32 = jnp.zeros((n, d), jnp.float32)
    z16 = jnp.zeros((n, 16), jnp.float32)
    ones = jnp.ones((ch, 16), jnp.float32)

    @functools.partial(
        pl.kernel,
        out_type=(
            jax.ShapeDtypeStruct((nc, n, d), jnp.float32),
            jax.ShapeDtypeStruct((nc, n, 16), jnp.float32),
        ),
        mesh=mesh,
        compiler_params=pltpu.CompilerParams(use_tc_tiling_on_sc=False),
        scratch_types=[
            pltpu.VMEM((base_rows + 1, ch), jnp.int32),
            pltpu.VMEM((10 * ch, d), jnp.float32),
            pltpu.VMEM((10 * ch, d), jnp.float32),
            pltpu.VMEM((ch, 16), jnp.float32),
            pltpu.VMEM_SHARED((n, d), jnp.float32),
            pltpu.VMEM_SHARED((n, 16), jnp.float32),
            pltpu.SemaphoreType.DMA,
        ],
    )
    def scatter_k(msg_hbm, dst_hbm, z32_hbm, z16_hbm, ones_hbm,
                  agg_hbm, deg_hbm, idx_v, rows0, rows1, ones_v,
                  agg_sh, deg_sh, asem):
        c = lax.axis_index("c")
        s = lax.axis_index("s")
        w = s * nc + c
        start, _, has_extra = _worker_rows(w, nch, nw)

        @pl.when(s == 0)
        def _init():
            pltpu.sync_copy(z32_hbm, agg_sh)
            pltpu.sync_copy(z16_hbm, deg_sh)

        pltpu.sync_copy(ones_hbm, ones_v)
        pltpu.sync_copy(dst_hbm.at[pl.ds(start, base_rows)],
                        idx_v.at[pl.ds(0, base_rows)])

        @pl.when(has_extra)
        def _():
            pltpu.sync_copy(dst_hbm.at[pl.ds(start + base_rows, 1)],
                            idx_v.at[pl.ds(base_rows, 1)])

        plsc.subcore_barrier()

        rows = (rows0, rows1)
        adds = {}
        for r in range(nrounds):
            buf = rows[r % 2]
            if r >= 2:
                for dsc in adds[r - 2]:
                    dsc.wait()
            jlo = r * 10
            jhi = min(jlo + 10, base_rows)
            nfull = jhi - jlo
            pltpu.sync_copy(
                msg_hbm.at[pl.ds((start + jlo) * ch, nfull * ch)],
                buf.at[pl.ds(0, nfull * ch)])
            descs = []
            for j in range(jlo, jhi):
                descs.append(pltpu.async_copy(
                    buf.at[pl.ds((j - jlo) * ch, ch)],
                    agg_sh.at[idx_v.at[j]], asem, add=True))
                descs.append(pltpu.async_copy(
                    ones_v, deg_sh.at[idx_v.at[j]], asem, add=True))
            if jhi == base_rows:
                @pl.when(has_extra)
                def _():
                    pltpu.sync_copy(
                        msg_hbm.at[pl.ds((start + base_rows) * ch, ch)],
                        buf.at[pl.ds(nfull * ch, ch)])
                    pltpu.sync_copy(buf.at[pl.ds(nfull * ch, ch)],
                                    agg_sh.at[idx_v.at[base_rows]], add=True)
                    pltpu.sync_copy(ones_v,
                                    deg_sh.at[idx_v.at[base_rows]], add=True)
            adds[r] = descs
        for r in (nrounds - 2, nrounds - 1):
            if r >= 0:
                for dsc in adds[r]:
                    dsc.wait()

        plsc.subcore_barrier()

        @pl.when(s == 0)
        def _emit():
            pltpu.sync_copy(agg_sh, agg_hbm.at[c])
            pltpu.sync_copy(deg_sh, deg_hbm.at[c])

    return scatter_k(msg, dst2d, z32, z16, ones)


def _tc_messages(ea, xj, w1t, b1, w2t, b2):
    e, k_in = ea.shape
    d = xj.shape[1]
    hidden = w1t.shape[1]
    be = 4000
    assert e % be == 0
    grid = (e // be,)
    rep = (jnp.arange(d)[:, None] == (jnp.arange(d * d)[None, :] // d))
    rep = rep.astype(jnp.bfloat16)
    b2mat = b2.reshape(d, d).astype(jnp.float32)

    def body(ea_ref, xj_ref, w1_ref, b1_ref, w2_ref, b2_ref, rep_ref,
             out_ref):
        h = jnp.maximum(
            jnp.dot(ea_ref[...].astype(jnp.bfloat16), w1_ref[...],
                    preferred_element_type=jnp.float32) + b1_ref[...], 0.0)
        we = jnp.dot(h.astype(jnp.bfloat16), w2_ref[...],
                     preferred_element_type=jnp.float32)
        xjr = jnp.dot(xj_ref[...].astype(jnp.bfloat16), rep_ref[...],
                      preferred_element_type=jnp.float32)
        acc = jnp.zeros((be, 128), jnp.float32)
        for j in range(d * d // 128):
            acc = acc + (xjr[:, j * 128:(j + 1) * 128]
                         * we[:, j * 128:(j + 1) * 128])
        msg = jnp.dot(xj_ref[...], b2_ref[...],
                      preferred_element_type=jnp.float32)
        msg = msg + ((acc[:, 0:d] + acc[:, d:2 * d])
                     + (acc[:, 2 * d:3 * d] + acc[:, 3 * d:4 * d]))
        out_ref[...] = msg

    return pl.pallas_call(
        body,
        grid=grid,
        in_specs=[
            pl.BlockSpec((be, k_in), lambda i: (i, 0)),
            pl.BlockSpec((be, d), lambda i: (i, 0)),
            pl.BlockSpec((k_in, hidden), lambda i: (0, 0)),
            pl.BlockSpec((1, hidden), lambda i: (0, 0)),
            pl.BlockSpec((hidden, d * d), lambda i: (0, 0)),
            pl.BlockSpec((d, d), lambda i: (0, 0)),
            pl.BlockSpec((d, d * d), lambda i: (0, 0)),
        ],
        out_specs=pl.BlockSpec((be, d), lambda i: (i, 0)),
        out_shape=jax.ShapeDtypeStruct((e, d), jnp.float32),
    )(ea.astype(jnp.float32), xj, w1t.astype(jnp.bfloat16),
      b1.astype(jnp.float32), w2t.astype(jnp.bfloat16), b2mat,
      rep)


def _tc_finalize(x, agg_p, deg_p, batch2, root, conv_b,
                 gn_w, gn_b, gn_ms, num_graphs, eps):
    n, d = x.shape
    bn = 1000
    assert n % bn == 0
    grid = (n // bn,)

    def stats_body(x_ref, agg_ref, deg_ref, bat_ref, root_ref, cb_ref,
                   h2_ref, cnt_ref, s_ref, ss_ref):
        @pl.when(pl.program_id(0) == 0)
        def _init():
            cnt_ref[...] = jnp.zeros_like(cnt_ref)
            s_ref[...] = jnp.zeros_like(s_ref)
            ss_ref[...] = jnp.zeros_like(ss_ref)

        x = x_ref[...]
        agg_sum = agg_ref[0] + agg_ref[1]
        deg = deg_ref[0][:, 0:1] + deg_ref[1][:, 0:1]
        agg = agg_sum / jnp.maximum(deg, 1.0)
        conv = agg + jnp.dot(x, root_ref[...],
                             preferred_element_type=jnp.float32) + cb_ref[...]
        h2 = jnp.maximum(conv, 0.0)
        h2_ref[...] = h2
        bat = bat_ref[...]
        oh = (bat == lax.broadcasted_iota(jnp.int32, (1, num_graphs), 1))
        oh = oh.astype(jnp.float32)
        ones_col = jnp.ones((bn, 1), jnp.float32)
        hcat = jnp.concatenate([h2, h2 * h2, jnp.broadcast_to(ones_col, (bn, d))],
                               axis=1)
        acc = lax.dot_general(oh, hcat, (((0,), (0,)), ((), ())),
                              preferred_element_type=jnp.float32)
        s_ref[...] += acc[:, 0:d]
        ss_ref[...] += acc[:, d:2 * d]
        cnt_ref[...] += acc[:, 2 * d:3 * d]

    h2, cnts, sums, sqs = pl.pallas_call(
        stats_body,
        grid=grid,
        in_specs=[
            pl.BlockSpec((bn, d), lambda i: (i, 0)),
            pl.BlockSpec((2, bn, d), lambda i: (0, i, 0)),
            pl.BlockSpec((2, bn, 16), lambda i: (0, i, 0)),
            pl.BlockSpec((bn, 1), lambda i: (i, 0)),
            pl.BlockSpec((d, d), lambda i: (0, 0)),
            pl.BlockSpec((1, d), lambda i: (0, 0)),
        ],
        out_specs=[
            pl.BlockSpec((bn, d), lambda i: (i, 0)),
            pl.BlockSpec((num_graphs, d), lambda i: (0, 0)),
            pl.BlockSpec((num_graphs, d), lambda i: (0, 0)),
            pl.BlockSpec((num_graphs, d), lambda i: (0, 0)),
        ],
        out_shape=[
            jax.ShapeDtypeStruct((n, d), jnp.float32),
            jax.ShapeDtypeStruct((num_graphs, d), jnp.float32),
            jax.ShapeDtypeStruct((num_graphs, d), jnp.float32),
            jax.ShapeDtypeStruct((num_graphs, d), jnp.float32),
        ],
    )(x, agg_p, deg_p, batch2, root, conv_b)

    def norm_body(h2_ref, x_ref, bat_ref, cnt_ref, s_ref, ss_ref,
                  gw_ref, gb_ref, gms_ref, out_ref):
        cnt = jnp.maximum(cnt_ref[...], 1.0)
        gms = gms_ref[...]
        mean = s_ref[...] / cnt
        var = ss_ref[...] / cnt - mean * mean * gms * (2.0 - gms)
        bat = bat_ref[...]
        h2 = h2_ref[...]
        oh = (bat == lax.broadcasted_iota(jnp.int32, (1, num_graphs), 1))
        oh = oh.astype(jnp.float32)
        mean_b = jnp.dot(oh, mean, preferred_element_type=jnp.float32)
        var_b = jnp.dot(oh, var, preferred_element_type=jnp.float32)
        cen = h2 - mean_b * gms
        normed = gw_ref[...] * cen * lax.rsqrt(var_b + eps) + gb_ref[...]
        out_ref[...] = normed + x_ref[...]

    return pl.pallas_call(
        norm_body,
        grid=grid,
        in_specs=[
            pl.BlockSpec((bn, d), lambda i: (i, 0)),
            pl.BlockSpec((bn, d), lambda i: (i, 0)),
            pl.BlockSpec((bn, 1), lambda i: (i, 0)),
            pl.BlockSpec((num_graphs, d), lambda i: (0, 0)),
            pl.BlockSpec((num_graphs, d), lambda i: (0, 0)),
            pl.BlockSpec((num_graphs, d), lambda i: (0, 0)),
            pl.BlockSpec((1, d), lambda i: (0, 0)),
            pl.BlockSpec((1, d), lambda i: (0, 0)),
            pl.BlockSpec((1, d), lambda i: (0, 0)),
        ],
        out_specs=pl.BlockSpec((bn, d), lambda i: (i, 0)),
        out_shape=jax.ShapeDtypeStruct((n, d), jnp.float32),
    )(h2, x, batch2, cnts, sums, sqs, gn_w, gn_b, gn_ms)


def kernel(x, edge_index, edge_attr, batch, W1, b1, W2, b2, root, conv_bias,
           gn_weight, gn_bias, gn_mean_scale):
    n, d = x.shape
    num_graphs = 8
    eps = 1e-5
    e = edge_attr.shape[0]
    src2d = edge_index[0].reshape(e // _CH, _CH)
    dst2d = edge_index[1].reshape(e // _CH, _CH)
    xj = _sc_gather(x, src2d)
    msg = _tc_messages(edge_attr, xj, W1.T, b1.reshape(1, -1), W2.T,
                       b2.reshape(1, -1))
    agg_p, deg_p = _sc_scatter(msg, dst2d, n)
    return _tc_finalize(x, agg_p, deg_p, batch.reshape(n, 1),
                        root, conv_bias.reshape(1, -1),
                        gn_weight.reshape(1, -1), gn_bias.reshape(1, -1),
                        gn_mean_scale.reshape(1, -1), num_graphs, eps)


# speedup vs baseline: 1.1054x; 1.0180x over previous
import functools

import jax
import jax.numpy as jnp
from jax import lax
from jax.experimental import pallas as pl
from jax.experimental.pallas import tpu as pltpu
from jax.experimental.pallas import tpu_sc as plsc

_CH = 128


def _sc_info():
    info = plsc.get_sparse_core_info()
    return info.num_cores, info.num_subcores


def _worker_rows(w, nch, nw):
    extra = nch % nw
    base_rows = nch // nw
    start = w * base_rows + jnp.minimum(w, extra)
    return start, base_rows, w < extra


def _sc_gather(x, src2d):
    n, d = x.shape
    nch, ch = src2d.shape
    e = nch * ch
    nc, ns = _sc_info()
    nw = nc * ns
    base_rows = nch // nw
    nrounds = (base_rows + 10) // 10
    mesh = plsc.VectorSubcoreMesh(core_axis_name="c", subcore_axis_name="s")

    @functools.partial(
        pl.kernel,
        out_type=jax.ShapeDtypeStruct((e, d), jnp.float32),
        mesh=mesh,
        compiler_params=pltpu.CompilerParams(use_tc_tiling_on_sc=False),
        scratch_types=[
            pltpu.VMEM((base_rows + 1, ch), jnp.int32),
            pltpu.VMEM((10 * ch, d), jnp.float32),
            pltpu.VMEM((10 * ch, d), jnp.float32),
            pltpu.VMEM_SHARED((n, d), jnp.float32),
            pltpu.SemaphoreType.DMA,
            pltpu.SemaphoreType.DMA,
        ],
    )
    def gather_k(x_hbm, src_hbm, out_hbm, idx_v, rows0, rows1, xs_sh,
                 gsem, wsem):
        c = lax.axis_index("c")
        s = lax.axis_index("s")
        w = s * nc + c
        start, _, has_extra = _worker_rows(w, nch, nw)

        @pl.when(s == 0)
        def _stage():
            pltpu.sync_copy(x_hbm, xs_sh)

        pltpu.sync_copy(src_hbm.at[pl.ds(start, base_rows)],
                        idx_v.at[pl.ds(0, base_rows)])

        @pl.when(has_extra)
        def _():
            pltpu.sync_copy(src_hbm.at[pl.ds(start + base_rows, 1)],
                            idx_v.at[pl.ds(base_rows, 1)])

        plsc.subcore_barrier()
        rows = (rows0, rows1)
        wb = {}
        for r in range(nrounds):
            buf = rows[r % 2]
            if r >= 2:
                for dsc in wb[r - 2]:
                    dsc.wait()
            jlo = r * 10
            jhi = min(jlo + 10, base_rows)
            descs = []
            for j in range(jlo, jhi):
                descs.append(pltpu.async_copy(
                    xs_sh.at[idx_v.at[j]],
                    buf.at[pl.ds((j - jlo) * ch, ch)], gsem))
            if jhi == base_rows:
                @pl.when(has_extra)
                def _():
                    pltpu.async_copy(
                        xs_sh.at[idx_v.at[base_rows]],
                        buf.at[pl.ds((base_rows - jlo) * ch, ch)],
                        gsem).wait()
            for dsc in descs:
                dsc.wait()
            nfull = jhi - jlo
            wbs = [pltpu.async_copy(
                buf.at[pl.ds(0, nfull * ch)],
                out_hbm.at[pl.ds((start + jlo) * ch, nfull * ch)], wsem)]
            if jhi == base_rows:
                @pl.when(has_extra)
                def _():
                    pltpu.async_copy(
                        buf.at[pl.ds(nfull * ch, ch)],
                        out_hbm.at[pl.ds((start + base_rows) * ch, ch)],
                        wsem).wait()
            wb[r] = wbs
        for r in (nrounds - 2, nrounds - 1):
            if r >= 0:
                for dsc in wb[r]:
                    dsc.wait()

    return gather_k(x, src2d)


def _sc_scatter(msg, dst2d, n):
    e, d = msg.shape
    nch, ch = dst2d.shape
    assert nch * ch == e
    nc, ns = _sc_info()
    nw = nc * ns
    base_rows = nch // nw
    nrounds = (base_rows + 10) // 10
    mesh = plsc.VectorSubcoreMesh(core_axis_name="c", subcore_axis_name="s")

    z32 = jnp.zeros((n, d), jnp.float32)
    z16 = jnp.zeros((n, 16), jnp.float32)
    ones = jnp.ones((ch, 16), jnp.float32)

    @functools.partial(
        pl.kernel,
        out_type=(
            jax.ShapeDtypeStruct((nc, n, d), jnp.float32),
            jax.ShapeDtypeStruct((nc, n, 16), jnp.float32),
        ),
        mesh=mesh,
        compiler_params=pltpu.CompilerParams(use_tc_tiling_on_sc=False),
        scratch_types=[
            pltpu.VMEM((base_rows + 1, ch), jnp.int32),
            pltpu.VMEM((10 * ch, d), jnp.float32),
            pltpu.VMEM((10 * ch, d), jnp.float32),
            pltpu.VMEM((ch, 16), jnp.float32),
            pltpu.VMEM_SHARED((n, d), jnp.float32),
            pltpu.VMEM_SHARED((n, 16), jnp.float32),
            pltpu.SemaphoreType.DMA,
        ],
    )
    def scatter_k(msg_hbm, dst_hbm, z32_hbm, z16_hbm, ones_hbm,
                  agg_hbm, deg_hbm, idx_v, rows0, rows1, ones_v,
                  agg_sh, deg_sh, asem):
        c = lax.axis_index("c")
        s = lax.axis_index("s")
        w = s * nc + c
        start, _, has_extra = _worker_rows(w, nch, nw)

        @pl.when(s == 0)
        def _init():
            pltpu.sync_copy(z32_hbm, agg_sh)
            pltpu.sync_copy(z16_hbm, deg_sh)

        pltpu.sync_copy(ones_hbm, ones_v)
        pltpu.sync_copy(dst_hbm.at[pl.ds(start, base_rows)],
                        idx_v.at[pl.ds(0, base_rows)])

        @pl.when(has_extra)
        def _():
            pltpu.sync_copy(dst_hbm.at[pl.ds(start + base_rows, 1)],
                            idx_v.at[pl.ds(base_rows, 1)])

        plsc.subcore_barrier()

        rows = (rows0, rows1)
        adds = {}
        for r in range(nrounds):
            buf = rows[r % 2]
            if r >= 2:
                for dsc in adds[r - 2]:
                    dsc.wait()
            jlo = r * 10
            jhi = min(jlo + 10, base_rows)
            nfull = jhi - jlo
            pltpu.sync_copy(
                msg_hbm.at[pl.ds((start + jlo) * ch, nfull * ch)],
                buf.at[pl.ds(0, nfull * ch)])
            descs = []
            for j in range(jlo, jhi):
                descs.append(pltpu.async_copy(
                    buf.at[pl.ds((j - jlo) * ch, ch)],
                    agg_sh.at[idx_v.at[j]], asem, add=True))
                descs.append(pltpu.async_copy(
                    ones_v, deg_sh.at[idx_v.at[j]], asem, add=True))
            if jhi == base_rows:
                @pl.when(has_extra)
                def _():
                    pltpu.sync_copy(
                        msg_hbm.at[pl.ds((start + base_rows) * ch, ch)],
                        buf.at[pl.ds(nfull * ch, ch)])
                    pltpu.sync_copy(buf.at[pl.ds(nfull * ch, ch)],
                                    agg_sh.at[idx_v.at[base_rows]], add=True)
                    pltpu.sync_copy(ones_v,
                                    deg_sh.at[idx_v.at[base_rows]], add=True)
            adds[r] = descs
        for r in (nrounds - 2, nrounds - 1):
            if r >= 0:
                for dsc in adds[r]:
                    dsc.wait()

        plsc.subcore_barrier()

        @pl.when(s == 0)
        def _emit():
            pltpu.sync_copy(agg_sh, agg_hbm.at[c])
            pltpu.sync_copy(deg_sh, deg_hbm.at[c])

    return scatter_k(msg, dst2d, z32, z16, ones)


def _tc_messages(ea, xj, w1t, b1, w2t, b2):
    e, k_in = ea.shape
    d = xj.shape[1]
    hidden = w1t.shape[1]
    be = 4000
    assert e % be == 0
    grid = (e // be,)
    rep = (jnp.arange(d)[:, None] == (jnp.arange(d * d)[None, :] // d))
    rep = rep.astype(jnp.bfloat16)
    b2mat = b2.reshape(d, d).astype(jnp.float32)

    def body(ea_ref, xj_ref, w1_ref, b1_ref, w2_ref, b2_ref, rep_ref,
             out_ref):
        h = jnp.maximum(
            jnp.dot(ea_ref[...].astype(jnp.bfloat16), w1_ref[...],
                    preferred_element_type=jnp.float32) + b1_ref[...], 0.0)
        we = jnp.dot(h.astype(jnp.bfloat16), w2_ref[...],
                     preferred_element_type=jnp.float32)
        xjr = jnp.dot(xj_ref[...].astype(jnp.bfloat16), rep_ref[...],
                      preferred_element_type=jnp.float32)
        acc = jnp.zeros((be, 128), jnp.float32)
        for j in range(d * d // 128):
            acc = acc + (xjr[:, j * 128:(j + 1) * 128]
                         * we[:, j * 128:(j + 1) * 128])
        msg = jnp.dot(xj_ref[...], b2_ref[...],
                      preferred_element_type=jnp.float32)
        msg = msg + ((acc[:, 0:d] + acc[:, d:2 * d])
                     + (acc[:, 2 * d:3 * d] + acc[:, 3 * d:4 * d]))
        out_ref[...] = msg

    return pl.pallas_call(
        body,
        grid=grid,
        in_specs=[
            pl.BlockSpec((be, k_in), lambda i: (i, 0)),
            pl.BlockSpec((be, d), lambda i: (i, 0)),
            pl.BlockSpec((k_in, hidden), lambda i: (0, 0)),
            pl.BlockSpec((1, hidden), lambda i: (0, 0)),
            pl.BlockSpec((hidden, d * d), lambda i: (0, 0)),
            pl.BlockSpec((d, d), lambda i: (0, 0)),
            pl.BlockSpec((d, d * d), lambda i: (0, 0)),
        ],
        out_specs=pl.BlockSpec((be, d), lambda i: (i, 0)),
        out_shape=jax.ShapeDtypeStruct((e, d), jnp.float32),
    )(ea.astype(jnp.float32), xj, w1t.astype(jnp.bfloat16),
      b1.astype(jnp.float32), w2t.astype(jnp.bfloat16), b2mat,
      rep)


def _tc_finalize(x, agg_p, deg_p, batch2, root, conv_b,
                 gn_w, gn_b, gn_ms, num_graphs, eps):
    n, d = x.shape
    bn = 1000
    assert n % bn == 0
    grid = (n // bn,)

    def stats_body(x_ref, agg_ref, deg_ref, bat_ref, root_ref, cb_ref,
                   h2_ref, cnt_ref, s_ref, ss_ref):
        @pl.when(pl.program_id(0) == 0)
        def _init():
            cnt_ref[...] = jnp.zeros_like(cnt_ref)
            s_ref[...] = jnp.zeros_like(s_ref)
            ss_ref[...] = jnp.zeros_like(ss_ref)

        x = x_ref[...]
        agg_sum = agg_ref[0] + agg_ref[1]
        deg = deg_ref[0][:, 0:1] + deg_ref[1][:, 0:1]
        agg = agg_sum / jnp.maximum(deg, 1.0)
        conv = agg + jnp.dot(x, root_ref[...],
                             preferred_element_type=jnp.float32) + cb_ref[...]
        h2 = jnp.maximum(conv, 0.0)
        h2_ref[...] = h2
        bat = bat_ref[...]
        oh = (bat == lax.broadcasted_iota(jnp.int32, (1, num_graphs), 1))
        oh = oh.astype(jnp.float32)
        ones_col = jnp.ones((bn, 1), jnp.float32)
        hcat = jnp.concatenate([h2, h2 * h2, jnp.broadcast_to(ones_col, (bn, d))],
                               axis=1)
        acc = lax.dot_general(oh, hcat, (((0,), (0,)), ((), ())),
                              preferred_element_type=jnp.float32)
        s_ref[...] += acc[:, 0:d]
        ss_ref[...] += acc[:, d:2 * d]
        cnt_ref[...] += acc[:, 2 * d:3 * d]

    h2, cnts, sums, sqs = pl.pallas_call(
        stats_body,
        grid=grid,
        in_specs=[
            pl.BlockSpec((bn, d), lambda i: (i, 0)),
            pl.BlockSpec((2, bn, d), lambda i: (0, i, 0)),
            pl.BlockSpec((2, bn, 16), lambda i: (0, i, 0)),
            pl.BlockSpec((bn, 1), lambda i: (i, 0)),
            pl.BlockSpec((d, d), lambda i: (0, 0)),
            pl.BlockSpec((1, d), lambda i: (0, 0)),
        ],
        out_specs=[
            pl.BlockSpec((bn, d), lambda i: (i, 0)),
            pl.BlockSpec((num_graphs, d), lambda i: (0, 0)),
            pl.BlockSpec((num_graphs, d), lambda i: (0, 0)),
            pl.BlockSpec((num_graphs, d), lambda i: (0, 0)),
        ],
        out_shape=[
            jax.ShapeDtypeStruct((n, d), jnp.float32),
            jax.ShapeDtypeStruct((num_graphs, d), jnp.float32),
            jax.ShapeDtypeStruct((num_graphs, d), jnp.float32),
            jax.ShapeDtypeStruct((num_graphs, d), jnp.float32),
        ],
    )(x, agg_p, deg_p, batch2, root, conv_b)

    def norm_body(h2_ref, x_ref, bat_ref, cnt_ref, s_ref, ss_ref,
                  gw_ref, gb_ref, gms_ref, out_ref):
        cnt = jnp.maximum(cnt_ref[...], 1.0)
        gms = gms_ref[...]
        mean = s_ref[...] / cnt
        var = ss_ref[...] / cnt - mean * mean * gms * (2.0 - gms)
        bat = bat_ref[...]
        h2 = h2_ref[...]
        oh = (bat == lax.broadcasted_iota(jnp.int32, (1, num_graphs), 1))
        oh = oh.astype(jnp.float32)
        mean_b = jnp.dot(oh, mean, preferred_element_type=jnp.float32)
        var_b = jnp.dot(oh, var, preferred_element_type=jnp.float32)
        cen = h2 - mean_b * gms
        normed = gw_ref[...] * cen * lax.rsqrt(var_b + eps) + gb_ref[...]
        out_ref[...] = normed + x_ref[...]

    return pl.pallas_call(
        norm_body,
        grid=grid,
        in_specs=[
            pl.BlockSpec((bn, d), lambda i: (i, 0)),
            pl.BlockSpec((bn, d), lambda i: (i, 0)),
            pl.BlockSpec((bn, 1), lambda i: (i, 0)),
            pl.BlockSpec((num_graphs, d), lambda i: (0, 0)),
            pl.BlockSpec((num_graphs, d), lambda i: (0, 0)),
            pl.BlockSpec((num_graphs, d), lambda i: (0, 0)),
            pl.BlockSpec((1, d), lambda i: (0, 0)),
            pl.BlockSpec((1, d), lambda i: (0, 0)),
            pl.BlockSpec((1, d), lambda i: (0, 0)),
        ],
        out_specs=pl.BlockSpec((bn, d), lambda i: (i, 0)),
        out_shape=jax.ShapeDtypeStruct((n, d), jnp.float32),
    )(h2, x, batch2, cnts, sums, sqs, gn_w, gn_b, gn_ms)


def kernel(x, edge_index, edge_attr, batch, W1, b1, W2, b2, root, conv_bias,
           gn_weight, gn_bias, gn_mean_scale):
    n, d = x.shape
    num_graphs = 8
    eps = 1e-5
    e = edge_attr.shape[0]
    src2d = edge_index[0].reshape(e // _CH, _CH)
    dst2d = edge_index[1].reshape(e // _CH, _CH)
    xj = _sc_gather(x, src2d)
    msg = _tc_messages(edge_attr, xj, W1.T, b1.reshape(1, -1), W2.T,
                       b2.reshape(1, -1))
    agg_p, deg_p = _sc_scatter(msg, dst2d, n)
    return _tc_finalize(x, agg_p, deg_p, batch.reshape(n, 1),
                        root, conv_bias.reshape(1, -1),
                        gn_weight.reshape(1, -1), gn_bias.reshape(1, -1),
                        gn_mean_scale.reshape(1, -1), num_graphs, eps)


# speedup vs baseline: 1.1074x; 1.0018x over previous
import functools

import jax
import jax.numpy as jnp
from jax import lax
from jax.experimental import pallas as pl
from jax.experimental.pallas import tpu as pltpu
from jax.experimental.pallas import tpu_sc as plsc

_CH = 128


def _sc_info():
    info = plsc.get_sparse_core_info()
    return info.num_cores, info.num_subcores


def _worker_rows(w, nch, nw):
    extra = nch % nw
    base_rows = nch // nw
    start = w * base_rows + jnp.minimum(w, extra)
    return start, base_rows, w < extra


def _sc_gather(x, src2d):
    n, d = x.shape
    nch, ch = src2d.shape
    e = nch * ch
    nc, ns = _sc_info()
    nw = nc * ns
    base_rows = nch // nw
    nrounds = (base_rows + 10) // 10
    mesh = plsc.VectorSubcoreMesh(core_axis_name="c", subcore_axis_name="s")

    @functools.partial(
        pl.kernel,
        out_type=jax.ShapeDtypeStruct((e, d), jnp.float32),
        mesh=mesh,
        compiler_params=pltpu.CompilerParams(use_tc_tiling_on_sc=False),
        scratch_types=[
            pltpu.VMEM((base_rows + 1, ch), jnp.int32),
            pltpu.VMEM((10 * ch, d), jnp.float32),
            pltpu.VMEM((10 * ch, d), jnp.float32),
            pltpu.VMEM_SHARED((n, d), jnp.float32),
            pltpu.SemaphoreType.DMA,
            pltpu.SemaphoreType.DMA,
        ],
    )
    def gather_k(x_hbm, src_hbm, out_hbm, idx_v, rows0, rows1, xs_sh,
                 gsem, wsem):
        c = lax.axis_index("c")
        s = lax.axis_index("s")
        w = s * nc + c
        start, _, has_extra = _worker_rows(w, nch, nw)

        @pl.when(s == 0)
        def _stage():
            pltpu.sync_copy(x_hbm, xs_sh)

        pltpu.sync_copy(src_hbm.at[pl.ds(start, base_rows)],
                        idx_v.at[pl.ds(0, base_rows)])

        @pl.when(has_extra)
        def _():
            pltpu.sync_copy(src_hbm.at[pl.ds(start + base_rows, 1)],
                            idx_v.at[pl.ds(base_rows, 1)])

        plsc.subcore_barrier()
        rows = (rows0, rows1)
        wb = {}
        for r in range(nrounds):
            buf = rows[r % 2]
            if r >= 2:
                for dsc in wb[r - 2]:
                    dsc.wait()
            jlo = r * 10
            jhi = min(jlo + 10, base_rows)
            descs = []
            for j in range(jlo, jhi):
                descs.append(pltpu.async_copy(
                    xs_sh.at[idx_v.at[j]],
                    buf.at[pl.ds((j - jlo) * ch, ch)], gsem))
            if jhi == base_rows:
                @pl.when(has_extra)
                def _():
                    pltpu.async_copy(
                        xs_sh.at[idx_v.at[base_rows]],
                        buf.at[pl.ds((base_rows - jlo) * ch, ch)],
                        gsem).wait()
            for dsc in descs:
                dsc.wait()
            nfull = jhi - jlo
            wbs = [pltpu.async_copy(
                buf.at[pl.ds(0, nfull * ch)],
                out_hbm.at[pl.ds((start + jlo) * ch, nfull * ch)], wsem)]
            if jhi == base_rows:
                @pl.when(has_extra)
                def _():
                    pltpu.async_copy(
                        buf.at[pl.ds(nfull * ch, ch)],
                        out_hbm.at[pl.ds((start + base_rows) * ch, ch)],
                        wsem).wait()
            wb[r] = wbs
        for r in (nrounds - 2, nrounds - 1):
            if r >= 0:
                for dsc in wb[r]:
                    dsc.wait()

    return gather_k(x, src2d)


def _sc_scatter(msg, dst2d, n):
    e, d = msg.shape
    nch, ch = dst2d.shape
    assert nch * ch == e
    nc, ns = _sc_info()
    nw = nc * ns
    base_rows = nch // nw
    nrounds = (base_rows + 10) // 10
    mesh = plsc.VectorSubcoreMesh(core_axis_name="c", subcore_axis_name="s")

    z32 = jnp.zeros((n, d), jnp.float32)
    z16 = jnp.zeros((n, 16), jnp.float32)
    ones = jnp.ones((ch, 16), jnp.float32)

    @functools.partial(
        pl.kernel,
        out_type=(
            jax.ShapeDtypeStruct((nc, n, d), jnp.float32),
            jax.ShapeDtypeStruct((nc, n, 16), jnp.float32),
        ),
        mesh=mesh,
        compiler_params=pltpu.CompilerParams(use_tc_tiling_on_sc=False),
        scratch_types=[
            pltpu.VMEM((base_rows + 1, ch), jnp.int32),
            pltpu.VMEM((10 * ch, d), jnp.float32),
            pltpu.VMEM((10 * ch, d), jnp.float32),
            pltpu.VMEM((ch, 16), jnp.float32),
            pltpu.VMEM_SHARED((n, d), jnp.float32),
            pltpu.VMEM_SHARED((n, 16), jnp.float32),
            pltpu.SemaphoreType.DMA,
        ],
    )
    def scatter_k(msg_hbm, dst_hbm, z32_hbm, z16_hbm, ones_hbm,
                  agg_hbm, deg_hbm, idx_v, rows0, rows1, ones_v,
                  agg_sh, deg_sh, asem):
        c = lax.axis_index("c")
        s = lax.axis_index("s")
        w = s * nc + c
        start, _, has_extra = _worker_rows(w, nch, nw)

        @pl.when(s == 0)
        def _init():
            pltpu.sync_copy(z32_hbm, agg_sh)
            pltpu.sync_copy(z16_hbm, deg_sh)

        pltpu.sync_copy(ones_hbm, ones_v)
        pltpu.sync_copy(dst_hbm.at[pl.ds(start, base_rows)],
                        idx_v.at[pl.ds(0, base_rows)])

        @pl.when(has_extra)
        def _():
            pltpu.sync_copy(dst_hbm.at[pl.ds(start + base_rows, 1)],
                            idx_v.at[pl.ds(base_rows, 1)])

        plsc.subcore_barrier()

        rows = (rows0, rows1)
        adds = {}
        for r in range(nrounds):
            buf = rows[r % 2]
            if r >= 2:
                for dsc in adds[r - 2]:
                    dsc.wait()
            jlo = r * 10
            jhi = min(jlo + 10, base_rows)
            nfull = jhi - jlo
            pltpu.sync_copy(
                msg_hbm.at[pl.ds((start + jlo) * ch, nfull * ch)],
                buf.at[pl.ds(0, nfull * ch)])
            descs = []
            for j in range(jlo, jhi):
                descs.append(pltpu.async_copy(
                    buf.at[pl.ds((j - jlo) * ch, ch)],
                    agg_sh.at[idx_v.at[j]], asem, add=True))
                descs.append(pltpu.async_copy(
                    ones_v, deg_sh.at[idx_v.at[j]], asem, add=True))
            if jhi == base_rows:
                @pl.when(has_extra)
                def _():
                    pltpu.sync_copy(
                        msg_hbm.at[pl.ds((start + base_rows) * ch, ch)],
                        buf.at[pl.ds(nfull * ch, ch)])
                    pltpu.sync_copy(buf.at[pl.ds(nfull * ch, ch)],
                                    agg_sh.at[idx_v.at[base_rows]], add=True)
                    pltpu.sync_copy(ones_v,
                                    deg_sh.at[idx_v.at[base_rows]], add=True)
            adds[r] = descs
        for r in (nrounds - 2, nrounds - 1):
            if r >= 0:
                for dsc in adds[r]:
                    dsc.wait()

        plsc.subcore_barrier()

        @pl.when(s == 0)
        def _emit():
            pltpu.sync_copy(agg_sh, agg_hbm.at[c])
            pltpu.sync_copy(deg_sh, deg_hbm.at[c])

    return scatter_k(msg, dst2d, z32, z16, ones)


def _tc_messages(ea, xj, w1t, b1, w2t, b2):
    e, k_in = ea.shape
    d = xj.shape[1]
    hidden = w1t.shape[1]
    be = 4000
    assert e % be == 0
    grid = (e // be,)
    rep = (jnp.arange(d)[:, None] == (jnp.arange(d * d)[None, :] // d))
    rep = rep.astype(jnp.bfloat16)
    b2mat = b2.reshape(d, d).astype(jnp.float32)

    def body(ea_ref, xj_ref, w1_ref, b1_ref, w2_ref, b2_ref, rep_ref,
             out_ref):
        h = jnp.maximum(
            jnp.dot(ea_ref[...].astype(jnp.bfloat16), w1_ref[...],
                    preferred_element_type=jnp.float32) + b1_ref[...], 0.0)
        we = jnp.dot(h.astype(jnp.bfloat16), w2_ref[...],
                     preferred_element_type=jnp.float32)
        xjr = jnp.dot(xj_ref[...].astype(jnp.bfloat16), rep_ref[...],
                      preferred_element_type=jnp.float32)
        acc = jnp.zeros((be, 128), jnp.float32)
        for j in range(d * d // 128):
            acc = acc + (xjr[:, j * 128:(j + 1) * 128]
                         * we[:, j * 128:(j + 1) * 128])
        msg = jnp.dot(xj_ref[...], b2_ref[...],
                      preferred_element_type=jnp.float32)
        msg = msg + ((acc[:, 0:d] + acc[:, d:2 * d])
                     + (acc[:, 2 * d:3 * d] + acc[:, 3 * d:4 * d]))
        out_ref[...] = msg

    return pl.pallas_call(
        body,
        grid=grid,
        in_specs=[
            pl.BlockSpec((be, k_in), lambda i: (i, 0)),
            pl.BlockSpec((be, d), lambda i: (i, 0)),
            pl.BlockSpec((k_in, hidden), lambda i: (0, 0)),
            pl.BlockSpec((1, hidden), lambda i: (0, 0)),
            pl.BlockSpec((hidden, d * d), lambda i: (0, 0)),
            pl.BlockSpec((d, d), lambda i: (0, 0)),
            pl.BlockSpec((d, d * d), lambda i: (0, 0)),
        ],
        out_specs=pl.BlockSpec((be, d), lambda i: (i, 0)),
        out_shape=jax.ShapeDtypeStruct((e, d), jnp.float32),
    )(ea.astype(jnp.float32), xj, w1t.astype(jnp.bfloat16),
      b1.astype(jnp.float32), w2t.astype(jnp.bfloat16), b2mat,
      rep)


def _tc_finalize(x, agg_p, deg_p, batch2, root, conv_b,
                 gn_w, gn_b, gn_ms, num_graphs, eps):
    n, d = x.shape
    bn = 1000
    assert n % bn == 0
    grid = (2, n // bn)

    def body(x_ref, agg_ref, deg_ref, bat_ref, root_ref, cb_ref,
             gw_ref, gb_ref, gms_ref, out_ref, h2_s, st_s):
        p = pl.program_id(0)
        i = pl.program_id(1)
        bat = bat_ref[...]
        oh = (bat == lax.broadcasted_iota(jnp.int32, (1, num_graphs), 1))
        oh = oh.astype(jnp.float32)

        @pl.when((p == 0) & (i == 0))
        def _init():
            st_s[...] = jnp.zeros_like(st_s)

        @pl.when(p == 0)
        def _phase0():
            x = x_ref[...]
            agg_sum = agg_ref[0] + agg_ref[1]
            deg = deg_ref[0][:, 0:1] + deg_ref[1][:, 0:1]
            agg = agg_sum / jnp.maximum(deg, 1.0)
            conv = agg + jnp.dot(
                x, root_ref[...],
                preferred_element_type=jnp.float32) + cb_ref[...]
            h2 = jnp.maximum(conv, 0.0)
            h2_s[pl.ds(i * bn, bn), :] = h2
            ones_col = jnp.ones((bn, 1), jnp.float32)
            hcat = jnp.concatenate(
                [h2, h2 * h2, jnp.broadcast_to(ones_col, (bn, d))], axis=1)
            st_s[...] += lax.dot_general(oh, hcat, (((0,), (0,)), ((), ())),
                                         preferred_element_type=jnp.float32)

        @pl.when(p == 1)
        def _phase1():
            st = st_s[...]
            cnt = jnp.maximum(st[:, 2 * d:3 * d], 1.0)
            gms = gms_ref[...]
            mean = st[:, 0:d] / cnt
            var = st[:, d:2 * d] / cnt - mean * mean * gms * (2.0 - gms)
            h2 = h2_s[pl.ds(i * bn, bn), :]
            mean_b = jnp.dot(oh, mean, preferred_element_type=jnp.float32)
            var_b = jnp.dot(oh, var, preferred_element_type=jnp.float32)
            cen = h2 - mean_b * gms
            normed = gw_ref[...] * cen * lax.rsqrt(var_b + eps) + gb_ref[...]
            out_ref[...] = normed + x_ref[...]

    return pl.pallas_call(
        body,
        grid=grid,
        in_specs=[
            pl.BlockSpec((bn, d), lambda p, i: (i, 0)),
            pl.BlockSpec((2, bn, d),
                         lambda p, i: (0, jnp.where(p == 0, i, 0), 0)),
            pl.BlockSpec((2, bn, 16),
                         lambda p, i: (0, jnp.where(p == 0, i, 0), 0)),
            pl.BlockSpec((bn, 1), lambda p, i: (i, 0)),
            pl.BlockSpec((d, d), lambda p, i: (0, 0)),
            pl.BlockSpec((1, d), lambda p, i: (0, 0)),
            pl.BlockSpec((1, d), lambda p, i: (0, 0)),
            pl.BlockSpec((1, d), lambda p, i: (0, 0)),
            pl.BlockSpec((1, d), lambda p, i: (0, 0)),
        ],
        out_specs=pl.BlockSpec((bn, d),
                               lambda p, i: (jnp.where(p == 0, 0, i), 0)),
        out_shape=jax.ShapeDtypeStruct((n, d), jnp.float32),
        scratch_shapes=[
            pltpu.VMEM((n, d), jnp.float32),
            pltpu.VMEM((num_graphs, 3 * d), jnp.float32),
        ],
    )(x, agg_p, deg_p, batch2, root, conv_b, gn_w, gn_b, gn_ms)


def kernel(x, edge_index, edge_attr, batch, W1, b1, W2, b2, root, conv_bias,
           gn_weight, gn_bias, gn_mean_scale):
    n, d = x.shape
    num_graphs = 8
    eps = 1e-5
    e = edge_attr.shape[0]
    src2d = edge_index[0].reshape(e // _CH, _CH)
    dst2d = edge_index[1].reshape(e // _CH, _CH)
    xj = _sc_gather(x, src2d)
    msg = _tc_messages(edge_attr, xj, W1.T, b1.reshape(1, -1), W2.T,
                       b2.reshape(1, -1))
    agg_p, deg_p = _sc_scatter(msg, dst2d, n)
    return _tc_finalize(x, agg_p, deg_p, batch.reshape(n, 1),
                        root, conv_bias.reshape(1, -1),
                        gn_weight.reshape(1, -1), gn_bias.reshape(1, -1),
                        gn_mean_scale.reshape(1, -1), num_graphs, eps)
